# baseline (speedup 1.0000x reference)
import functools

import jax
import jax.numpy as jnp
from jax import lax
from jax.experimental import pallas as pl
from jax.experimental.pallas import tpu as pltpu
from jax.experimental.pallas import tpu_sc as plsc

B, S, H = 1, 2048, 768
FF = 3072
E = 8
K = 2
T = B * S

BLK = 256
NB = (K * T) // BLK + E
S_PAD = NB * BLK

NC, NS = 2, 16
NW = NC * NS
CHUNK = T // NW
LANES = 16


def _router_body(x_ref, wr_ref, br_ref,
                 pos1_ref, pos2_ref, w1_ref, w2_ref, be_ref, nb_ref, load_ref,
                 first_ref, par_ref, nxte_ref):
    x = x_ref[...]
    logits = jnp.dot(x, wr_ref[...], preferred_element_type=jnp.float32)
    logits = logits + br_ref[...]

    ie = lax.broadcasted_iota(jnp.int32, (T, E), 1)
    m1 = jnp.max(logits, axis=-1, keepdims=True)
    idx1 = jnp.min(jnp.where(logits == m1, ie, E), axis=-1, keepdims=True)
    masked = jnp.where(ie == idx1, -jnp.inf, logits)
    m2 = jnp.max(masked, axis=-1, keepdims=True)
    idx2 = jnp.min(jnp.where(masked == m2, ie, E), axis=-1, keepdims=True)

    e21 = jnp.exp(m2 - m1)
    w1 = 1.0 / (1.0 + e21)
    w2 = e21 / (1.0 + e21)
    w1_ref[...] = jnp.reshape(w1, (T,))
    w2_ref[...] = jnp.reshape(w2, (T,))

    oh = jnp.logical_or(ie == idx1, ie == idx2).astype(jnp.float32)
    counts = jnp.sum(oh, axis=0, keepdims=True)
    load_ref[...] = counts * (1.0 / T)

    c = oh
    sh = 1
    while sh < T:
        c = c + jnp.concatenate(
            [jnp.zeros((sh, E), jnp.float32), c[:T - sh, :]], axis=0)
        sh *= 2
    excl = c - oh

    r1 = jnp.sum(jnp.where(ie == idx1, excl, 0.0), axis=-1, keepdims=True)
    r2 = jnp.sum(jnp.where(ie == idx2, excl, 0.0), axis=-1, keepdims=True)

    counts_i = counts.astype(jnp.int32)
    bc = (counts_i + (BLK - 1)) // BLK
    cb = bc
    sh = 1
    while sh < E:
        cb = cb + jnp.concatenate(
            [jnp.zeros((1, sh), jnp.int32), cb[:, :E - sh]], axis=1)
        sh *= 2
    off_rows = (cb - bc) * BLK

    off_b = jnp.broadcast_to(off_rows, (T, E))
    o1 = jnp.sum(jnp.where(ie == idx1, off_b, 0), axis=-1, keepdims=True)
    o2 = jnp.sum(jnp.where(ie == idx2, off_b, 0), axis=-1, keepdims=True)
    pos1_ref[...] = jnp.reshape(o1 + r1.astype(jnp.int32), (T,))
    pos2_ref[...] = jnp.reshape(o2 + r2.astype(jnp.int32), (T,))

    ib = lax.broadcasted_iota(jnp.int32, (NB, E), 0)
    cb_b = jnp.broadcast_to(cb, (NB, E))
    be = jnp.sum((cb_b <= ib).astype(jnp.int32), axis=-1, keepdims=True)
    be = jnp.minimum(be, E - 1)
    be_ref[...] = jnp.reshape(be, (NB,))
    nb2 = jnp.max(cb, axis=-1, keepdims=True)
    nb_ref[...] = jnp.reshape(nb2, (1,))

    ibc = lax.broadcasted_iota(jnp.int32, (NB, 1), 0)
    be_prev = jnp.concatenate(
        [jnp.full((1, 1), -1, jnp.int32), be[:NB - 1, :]], axis=0)
    active = ibc < jnp.broadcast_to(nb2, (NB, 1))
    first = jnp.logical_and(be != be_prev, active).astype(jnp.int32)
    seg = first
    sh = 1
    while sh < NB:
        seg = seg + jnp.concatenate(
            [jnp.zeros((sh, 1), jnp.int32), seg[:NB - sh, :]], axis=0)
        sh *= 2
    first_ref[...] = jnp.reshape(first, (NB,))
    par_ref[...] = jnp.reshape(jnp.bitwise_and(seg + 1, 1), (NB,))
    ie_b = lax.broadcasted_iota(jnp.int32, (NB, E), 1)
    cnt_b = jnp.broadcast_to(counts_i, (NB, E))
    nxt_ok = jnp.logical_and(ie_b > jnp.broadcast_to(be, (NB, E)), cnt_b > 0)
    nxte_ref[...] = jnp.reshape(
        jnp.min(jnp.where(nxt_ok, ie_b, E), axis=-1, keepdims=True), (NB,))


def _router(x, W_r, b_r):
    return pl.pallas_call(
        _router_body,
        out_shape=(
            jax.ShapeDtypeStruct((T,), jnp.int32),
            jax.ShapeDtypeStruct((T,), jnp.int32),
            jax.ShapeDtypeStruct((T,), jnp.float32),
            jax.ShapeDtypeStruct((T,), jnp.float32),
            jax.ShapeDtypeStruct((NB,), jnp.int32),
            jax.ShapeDtypeStruct((1,), jnp.int32),
            jax.ShapeDtypeStruct((1, E), jnp.float32),
            jax.ShapeDtypeStruct((NB,), jnp.int32),
            jax.ShapeDtypeStruct((NB,), jnp.int32),
            jax.ShapeDtypeStruct((NB,), jnp.int32),
        ),
    )(x, W_r, b_r)


def _dispatch_body(x_hbm, pos1_hbm, pos2_hbm, wa_hbm, wb_hbm,
                   xs_hbm, ws_hbm,
                   rowsa_v, rowsb_v, i1a_v, i1b_v, i2a_v, i2b_v,
                   posf_v, wf_v, wbuf_v, sem):
    cid = lax.axis_index("c")
    sid = lax.axis_index("s")
    wid = sid * NC + cid
    base = wid * CHUNK
    half = CHUNK // 2

    pltpu.sync_copy(pos1_hbm.at[pl.ds(base, half)], i1a_v)
    pltpu.sync_copy(pos1_hbm.at[pl.ds(base + half, half)], i1b_v)
    pltpu.sync_copy(pos2_hbm.at[pl.ds(base, half)], i2a_v)
    pltpu.sync_copy(pos2_hbm.at[pl.ds(base + half, half)], i2b_v)
    pltpu.sync_copy(x_hbm.at[pl.ds(base, half)], rowsa_v)
    d1 = pltpu.async_copy(rowsa_v, xs_hbm.at[i1a_v], sem)
    d2 = pltpu.async_copy(rowsa_v, xs_hbm.at[i2a_v], sem)
    pltpu.sync_copy(x_hbm.at[pl.ds(base + half, half)], rowsb_v)
    d3 = pltpu.async_copy(rowsb_v, xs_hbm.at[i1b_v], sem)
    d4 = pltpu.async_copy(rowsb_v, xs_hbm.at[i2b_v], sem)
    d1.wait()
    d2.wait()
    d3.wait()
    d4.wait()

    @pl.when(wid == 0)
    def _():
        def scatter_half(pos_hbm, w_hbm):
            pltpu.sync_copy(pos_hbm, posf_v)
            pltpu.sync_copy(w_hbm, wf_v)

            def sc_body(j, _):
                idx = posf_v[pl.ds(j * LANES, LANES)]
                vals = wf_v[pl.ds(j * LANES, LANES)]
                plsc.store_scatter(wbuf_v, [idx], vals)
                return 0
            lax.fori_loop(0, T // LANES, sc_body, 0)

        scatter_half(pos1_hbm, wa_hbm)
        scatter_half(pos2_hbm, wb_hbm)
        pltpu.sync_copy(wbuf_v, ws_hbm)


def _dispatch(x, pos1, pos2, w1, w2):
    mesh = plsc.VectorSubcoreMesh(
        core_axis_name="c", subcore_axis_name="s", num_cores=NC, num_subcores=NS)
    fn = pl.kernel(
        _dispatch_body,
        out_type=(
            jax.ShapeDtypeStruct((S_PAD, H), jnp.float32),
            jax.ShapeDtypeStruct((S_PAD,), jnp.float32),
        ),
        mesh=mesh,
        scratch_types=[
            pltpu.VMEM((CHUNK // 2, H), jnp.float32),
            pltpu.VMEM((CHUNK // 2, H), jnp.float32),
            pltpu.VMEM((CHUNK // 2,), jnp.int32),
            pltpu.VMEM((CHUNK // 2,), jnp.int32),
            pltpu.VMEM((CHUNK // 2,), jnp.int32),
            pltpu.VMEM((CHUNK // 2,), jnp.int32),
            pltpu.VMEM((T,), jnp.int32),
            pltpu.VMEM((T,), jnp.float32),
            pltpu.VMEM((S_PAD,), jnp.float32),
            pltpu.SemaphoreType.DMA,
        ],
        compiler_params=pltpu.CompilerParams(needs_layout_passes=False),
    )
    return fn(x, pos1, pos2, w1, w2)


def _ffn_body(be_ref, nb_ref, first_ref, par_ref, nxte_ref,
              x_ref, w_ref, W1_hbm, b1_ref, W2_hbm, b2_ref, y_ref,
              w1buf, w2buf, sems):
    i = pl.program_id(0)

    def w_copies(e_idx, slot):
        hh = H // 2
        fh = FF // 2
        return (
            pltpu.make_async_copy(W1_hbm.at[e_idx, 0:hh], w1buf.at[slot, 0:hh],
                                  sems.at[slot, 0]),
            pltpu.make_async_copy(W1_hbm.at[e_idx, hh:H], w1buf.at[slot, hh:H],
                                  sems.at[slot, 0]),
            pltpu.make_async_copy(W2_hbm.at[e_idx, 0:fh], w2buf.at[slot, 0:fh],
                                  sems.at[slot, 1]),
            pltpu.make_async_copy(W2_hbm.at[e_idx, fh:FF], w2buf.at[slot, fh:FF],
                                  sems.at[slot, 1]),
        )

    @pl.when(i == 0)
    def _():
        for c in w_copies(be_ref[0], 0):
            c.start()

        @pl.when(nxte_ref[0] < E)
        def _():
            for c in w_copies(nxte_ref[0], 1):
                c.start()

    @pl.when(i < nb_ref[0])
    def _():
        p = par_ref[i]

        @pl.when(jnp.logical_and(first_ref[i] == 1, i > 0))
        def _():
            @pl.when(nxte_ref[i] < E)
            def _():
                for c in w_copies(nxte_ref[i], 1 - p):
                    c.start()

        @pl.when(first_ref[i] == 1)
        def _():
            for c in w_copies(be_ref[i], p):
                c.wait()

        x = x_ref[...]
        h = jnp.dot(x, w1buf[p], preferred_element_type=jnp.float32)
        h = h + b1_ref[0]
        h = 0.5 * h * (1.0 + lax.erf(h * 0.7071067811865476))
        o = jnp.dot(h, w2buf[p], preferred_element_type=jnp.float32)
        o = o + b2_ref[0]
        y_ref[...] = o * jnp.reshape(w_ref[...], (BLK, 1))


def _ffn(be, nb, first, par, nxte, xs, ws, W1, b1, W2, b2):
    grid_spec = pltpu.PrefetchScalarGridSpec(
        num_scalar_prefetch=5,
        grid=(NB,),
        in_specs=[
            pl.BlockSpec((BLK, H), lambda i, *sp: (i, 0)),
            pl.BlockSpec((BLK,), lambda i, *sp: (i,)),
            pl.BlockSpec(memory_space=pl.ANY),
            pl.BlockSpec((1, 1, FF), lambda i, be, nb, fi, pa, nx: (be[i], 0, 0)),
            pl.BlockSpec(memory_space=pl.ANY),
            pl.BlockSpec((1, 1, H), lambda i, be, nb, fi, pa, nx: (be[i], 0, 0)),
        ],
        out_specs=pl.BlockSpec((BLK, H), lambda i, *sp: (i, 0)),
        scratch_shapes=[
            pltpu.VMEM((2, H, FF), jnp.float32),
            pltpu.VMEM((2, FF, H), jnp.float32),
            pltpu.SemaphoreType.DMA((2, 2)),
        ],
    )
    return pl.pallas_call(
        _ffn_body,
        grid_spec=grid_spec,
        out_shape=jax.ShapeDtypeStruct((S_PAD, H), jnp.float32),
    )(be, nb, first, par, nxte,
      xs, ws, W1, b1.reshape(E, 1, FF), W2, b2.reshape(E, 1, H))


def _combine_body(y_hbm, pos1_hbm, pos2_hbm, out_hbm,
                  buf0_v, buf1_v, i1_v, i2_v, sem):
    cid = lax.axis_index("c")
    sid = lax.axis_index("s")
    wid = sid * NC + cid
    base = wid * CHUNK

    pltpu.sync_copy(pos1_hbm.at[pl.ds(base, CHUNK)], i1_v)
    pltpu.sync_copy(pos2_hbm.at[pl.ds(base, CHUNK)], i2_v)
    pltpu.async_copy(y_hbm.at[i1_v], buf0_v, sem).wait()
    pltpu.async_copy(y_hbm.at[i2_v], buf1_v, sem).wait()

    def row_body(r, _):
        for c in range(H // LANES):
            sl = pl.ds(c * LANES, LANES)
            buf0_v[r, sl] = buf0_v[r, sl] + buf1_v[r, sl]
        return 0
    lax.fori_loop(0, CHUNK, row_body, 0)

    pltpu.sync_copy(buf0_v, out_hbm.at[pl.ds(base, CHUNK)])


def _combine(y, pos1, pos2):
    mesh = plsc.VectorSubcoreMesh(
        core_axis_name="c", subcore_axis_name="s", num_cores=NC, num_subcores=NS)
    fn = pl.kernel(
        _combine_body,
        out_type=jax.ShapeDtypeStruct((T, H), jnp.float32),
        mesh=mesh,
        scratch_types=[
            pltpu.VMEM((CHUNK, H), jnp.float32),
            pltpu.VMEM((CHUNK, H), jnp.float32),
            pltpu.VMEM((CHUNK,), jnp.int32),
            pltpu.VMEM((CHUNK,), jnp.int32),
            pltpu.SemaphoreType.DMA,
        ],
    )
    return fn(y, pos1, pos2)


@jax.jit
def kernel(hidden_states, W_r, b_r, W1, b1, W2, b2):
    x = hidden_states.reshape(T, H)
    (pos1, pos2, w1, w2, be, nb, load,
     first, par, nxte) = _router(x, W_r, b_r.reshape(1, E))
    xs, ws = _dispatch(x, pos1, pos2, w1, w2)
    y = _ffn(be, nb, first, par, nxte, xs, ws, W1, b1, W2, b2)
    out = _combine(y, pos1, pos2)
    return out.reshape(B, S, H), load.reshape(E)


# speedup vs baseline: 1.0138x; 1.0138x over previous
import functools

import jax
import jax.numpy as jnp
from jax import lax
from jax.experimental import pallas as pl
from jax.experimental.pallas import tpu as pltpu
from jax.experimental.pallas import tpu_sc as plsc

B, S, H = 1, 2048, 768
FF = 3072
E = 8
K = 2
T = B * S

BLK = 256
NB = (K * T) // BLK + E
S_PAD = NB * BLK

NC, NS = 2, 16
NW = NC * NS
CHUNK = T // NW
LANES = 16


def _router_body(x_ref, wr_ref, br_ref,
                 pos1_ref, pos2_ref, w1_ref, w2_ref, be_ref, nb_ref, load_ref,
                 first_ref, par_ref, nxte_ref):
    x = x_ref[...]
    logits = jnp.dot(x, wr_ref[...], preferred_element_type=jnp.float32)
    logits = logits + br_ref[...]

    ie = lax.broadcasted_iota(jnp.int32, (T, E), 1)
    m1 = jnp.max(logits, axis=-1, keepdims=True)
    idx1 = jnp.min(jnp.where(logits == m1, ie, E), axis=-1, keepdims=True)
    masked = jnp.where(ie == idx1, -jnp.inf, logits)
    m2 = jnp.max(masked, axis=-1, keepdims=True)
    idx2 = jnp.min(jnp.where(masked == m2, ie, E), axis=-1, keepdims=True)

    e21 = jnp.exp(m2 - m1)
    w1 = 1.0 / (1.0 + e21)
    w2 = e21 / (1.0 + e21)
    w1_ref[...] = jnp.reshape(w1, (T,))
    w2_ref[...] = jnp.reshape(w2, (T,))

    oh = jnp.logical_or(ie == idx1, ie == idx2).astype(jnp.float32)
    counts = jnp.sum(oh, axis=0, keepdims=True)
    load_ref[...] = counts * (1.0 / T)

    c = oh
    sh = 1
    while sh < T:
        c = c + jnp.concatenate(
            [jnp.zeros((sh, E), jnp.float32), c[:T - sh, :]], axis=0)
        sh *= 2
    excl = c - oh

    r1 = jnp.sum(jnp.where(ie == idx1, excl, 0.0), axis=-1, keepdims=True)
    r2 = jnp.sum(jnp.where(ie == idx2, excl, 0.0), axis=-1, keepdims=True)

    counts_i = counts.astype(jnp.int32)
    bc = (counts_i + (BLK - 1)) // BLK
    cb = bc
    sh = 1
    while sh < E:
        cb = cb + jnp.concatenate(
            [jnp.zeros((1, sh), jnp.int32), cb[:, :E - sh]], axis=1)
        sh *= 2
    off_rows = (cb - bc) * BLK

    off_b = jnp.broadcast_to(off_rows, (T, E))
    o1 = jnp.sum(jnp.where(ie == idx1, off_b, 0), axis=-1, keepdims=True)
    o2 = jnp.sum(jnp.where(ie == idx2, off_b, 0), axis=-1, keepdims=True)
    pos1_ref[...] = jnp.reshape(o1 + r1.astype(jnp.int32), (T,))
    pos2_ref[...] = jnp.reshape(o2 + r2.astype(jnp.int32), (T,))

    ib = lax.broadcasted_iota(jnp.int32, (NB, E), 0)
    cb_b = jnp.broadcast_to(cb, (NB, E))
    be = jnp.sum((cb_b <= ib).astype(jnp.int32), axis=-1, keepdims=True)
    be = jnp.minimum(be, E - 1)
    be_ref[...] = jnp.reshape(be, (NB,))
    nb2 = jnp.max(cb, axis=-1, keepdims=True)
    nb_ref[...] = jnp.reshape(nb2, (1,))

    ibc = lax.broadcasted_iota(jnp.int32, (NB, 1), 0)
    be_prev = jnp.concatenate(
        [jnp.full((1, 1), -1, jnp.int32), be[:NB - 1, :]], axis=0)
    active = ibc < jnp.broadcast_to(nb2, (NB, 1))
    first = jnp.logical_and(be != be_prev, active).astype(jnp.int32)
    seg = first
    sh = 1
    while sh < NB:
        seg = seg + jnp.concatenate(
            [jnp.zeros((sh, 1), jnp.int32), seg[:NB - sh, :]], axis=0)
        sh *= 2
    first_ref[...] = jnp.reshape(first, (NB,))
    par_ref[...] = jnp.reshape(jnp.bitwise_and(seg + 1, 1), (NB,))
    ie_b = lax.broadcasted_iota(jnp.int32, (NB, E), 1)
    cnt_b = jnp.broadcast_to(counts_i, (NB, E))
    nxt_ok = jnp.logical_and(ie_b > jnp.broadcast_to(be, (NB, E)), cnt_b > 0)
    nxte_ref[...] = jnp.reshape(
        jnp.min(jnp.where(nxt_ok, ie_b, E), axis=-1, keepdims=True), (NB,))


def _router(x, W_r, b_r):
    return pl.pallas_call(
        _router_body,
        out_shape=(
            jax.ShapeDtypeStruct((T,), jnp.int32),
            jax.ShapeDtypeStruct((T,), jnp.int32),
            jax.ShapeDtypeStruct((T,), jnp.float32),
            jax.ShapeDtypeStruct((T,), jnp.float32),
            jax.ShapeDtypeStruct((NB,), jnp.int32),
            jax.ShapeDtypeStruct((1,), jnp.int32),
            jax.ShapeDtypeStruct((1, E), jnp.float32),
            jax.ShapeDtypeStruct((NB,), jnp.int32),
            jax.ShapeDtypeStruct((NB,), jnp.int32),
            jax.ShapeDtypeStruct((NB,), jnp.int32),
        ),
    )(x, W_r, b_r)


def _dispatch_body(x_hbm, pos1_hbm, pos2_hbm, wa_hbm, wb_hbm,
                   xs_hbm, ws_hbm,
                   rows_v, i1_v, i2_v, posf_v, wf_v, wbuf_v, sem):
    cid = lax.axis_index("c")
    sid = lax.axis_index("s")
    wid = sid * NC + cid
    base = wid * CHUNK

    pltpu.sync_copy(pos1_hbm.at[pl.ds(base, CHUNK)], i1_v)
    pltpu.sync_copy(pos2_hbm.at[pl.ds(base, CHUNK)], i2_v)
    pltpu.sync_copy(x_hbm.at[pl.ds(base, CHUNK)], rows_v)
    d1 = pltpu.async_copy(rows_v, xs_hbm.at[i1_v], sem)
    d2 = pltpu.async_copy(rows_v, xs_hbm.at[i2_v], sem)
    d1.wait()
    d2.wait()

    @pl.when(wid == 0)
    def _():
        def scatter_half(pos_hbm, w_hbm):
            pltpu.sync_copy(pos_hbm, posf_v)
            pltpu.sync_copy(w_hbm, wf_v)

            def sc_body(j, _):
                idx = posf_v[pl.ds(j * LANES, LANES)]
                vals = wf_v[pl.ds(j * LANES, LANES)]
                plsc.store_scatter(wbuf_v, [idx], vals)
                return 0
            lax.fori_loop(0, T // LANES, sc_body, 0)

        scatter_half(pos1_hbm, wa_hbm)
        scatter_half(pos2_hbm, wb_hbm)
        pltpu.sync_copy(wbuf_v, ws_hbm)


def _dispatch(x, pos1, pos2, w1, w2):
    mesh = plsc.VectorSubcoreMesh(
        core_axis_name="c", subcore_axis_name="s", num_cores=NC, num_subcores=NS)
    fn = pl.kernel(
        _dispatch_body,
        out_type=(
            jax.ShapeDtypeStruct((S_PAD, H), jnp.float32),
            jax.ShapeDtypeStruct((S_PAD,), jnp.float32),
        ),
        mesh=mesh,
        scratch_types=[
            pltpu.VMEM((CHUNK, H), jnp.float32),
            pltpu.VMEM((CHUNK,), jnp.int32),
            pltpu.VMEM((CHUNK,), jnp.int32),
            pltpu.VMEM((T,), jnp.int32),
            pltpu.VMEM((T,), jnp.float32),
            pltpu.VMEM((S_PAD,), jnp.float32),
            pltpu.SemaphoreType.DMA,
        ],
        compiler_params=pltpu.CompilerParams(needs_layout_passes=False),
    )
    return fn(x, pos1, pos2, w1, w2)


def _ffn_body(be_ref, nb_ref, first_ref, par_ref, nxte_ref,
              x_ref, w_ref, W1_hbm, b1_ref, W2_hbm, b2_ref, y_ref,
              w1buf, w2buf, sems):
    i = pl.program_id(0)

    def w_copies(e_idx, slot):
        hh = H // 2
        fh = FF // 2
        return (
            pltpu.make_async_copy(W1_hbm.at[e_idx, 0:hh], w1buf.at[slot, 0:hh],
                                  sems.at[slot, 0]),
            pltpu.make_async_copy(W1_hbm.at[e_idx, hh:H], w1buf.at[slot, hh:H],
                                  sems.at[slot, 0]),
            pltpu.make_async_copy(W2_hbm.at[e_idx, 0:fh], w2buf.at[slot, 0:fh],
                                  sems.at[slot, 1]),
            pltpu.make_async_copy(W2_hbm.at[e_idx, fh:FF], w2buf.at[slot, fh:FF],
                                  sems.at[slot, 1]),
        )

    @pl.when(i == 0)
    def _():
        for c in w_copies(be_ref[0], 0):
            c.start()

        @pl.when(nxte_ref[0] < E)
        def _():
            for c in w_copies(nxte_ref[0], 1):
                c.start()

    @pl.when(i < nb_ref[0])
    def _():
        p = par_ref[i]

        @pl.when(jnp.logical_and(first_ref[i] == 1, i > 0))
        def _():
            @pl.when(nxte_ref[i] < E)
            def _():
                for c in w_copies(nxte_ref[i], 1 - p):
                    c.start()

        @pl.when(first_ref[i] == 1)
        def _():
            for c in w_copies(be_ref[i], p):
                c.wait()

        x = x_ref[...]
        h = jnp.dot(x, w1buf[p], preferred_element_type=jnp.float32)
        h = h + b1_ref[0]
        h = 0.5 * h * (1.0 + lax.erf(h * 0.7071067811865476))
        o = jnp.dot(h, w2buf[p], preferred_element_type=jnp.float32)
        o = o + b2_ref[0]
        y_ref[...] = o * jnp.reshape(w_ref[...], (BLK, 1))


def _ffn(be, nb, first, par, nxte, xs, ws, W1, b1, W2, b2):
    grid_spec = pltpu.PrefetchScalarGridSpec(
        num_scalar_prefetch=5,
        grid=(NB,),
        in_specs=[
            pl.BlockSpec((BLK, H), lambda i, *sp: (i, 0)),
            pl.BlockSpec((BLK,), lambda i, *sp: (i,)),
            pl.BlockSpec(memory_space=pl.ANY),
            pl.BlockSpec((1, 1, FF), lambda i, be, nb, fi, pa, nx: (be[i], 0, 0)),
            pl.BlockSpec(memory_space=pl.ANY),
            pl.BlockSpec((1, 1, H), lambda i, be, nb, fi, pa, nx: (be[i], 0, 0)),
        ],
        out_specs=pl.BlockSpec((BLK, H), lambda i, *sp: (i, 0)),
        scratch_shapes=[
            pltpu.VMEM((2, H, FF), jnp.float32),
            pltpu.VMEM((2, FF, H), jnp.float32),
            pltpu.SemaphoreType.DMA((2, 2)),
        ],
    )
    return pl.pallas_call(
        _ffn_body,
        grid_spec=grid_spec,
        out_shape=jax.ShapeDtypeStruct((S_PAD, H), jnp.float32),
    )(be, nb, first, par, nxte,
      xs, ws, W1, b1.reshape(E, 1, FF), W2, b2.reshape(E, 1, H))


def _combine_body(y_hbm, pos1_hbm, pos2_hbm, out_hbm,
                  buf0_v, buf1_v, i1_v, i2_v, sem):
    cid = lax.axis_index("c")
    sid = lax.axis_index("s")
    wid = sid * NC + cid
    base = wid * CHUNK

    pltpu.sync_copy(pos1_hbm.at[pl.ds(base, CHUNK)], i1_v)
    pltpu.sync_copy(pos2_hbm.at[pl.ds(base, CHUNK)], i2_v)
    pltpu.async_copy(y_hbm.at[i1_v], buf0_v, sem).wait()
    pltpu.async_copy(y_hbm.at[i2_v], buf1_v, sem).wait()

    def row_body(r, _):
        for c in range(H // LANES):
            sl = pl.ds(c * LANES, LANES)
            buf0_v[r, sl] = buf0_v[r, sl] + buf1_v[r, sl]
        return 0
    lax.fori_loop(0, CHUNK, row_body, 0)

    pltpu.sync_copy(buf0_v, out_hbm.at[pl.ds(base, CHUNK)])


def _combine(y, pos1, pos2):
    mesh = plsc.VectorSubcoreMesh(
        core_axis_name="c", subcore_axis_name="s", num_cores=NC, num_subcores=NS)
    fn = pl.kernel(
        _combine_body,
        out_type=jax.ShapeDtypeStruct((T, H), jnp.float32),
        mesh=mesh,
        scratch_types=[
            pltpu.VMEM((CHUNK, H), jnp.float32),
            pltpu.VMEM((CHUNK, H), jnp.float32),
            pltpu.VMEM((CHUNK,), jnp.int32),
            pltpu.VMEM((CHUNK,), jnp.int32),
            pltpu.SemaphoreType.DMA,
        ],
    )
    return fn(y, pos1, pos2)


@jax.jit
def kernel(hidden_states, W_r, b_r, W1, b1, W2, b2):
    x = hidden_states.reshape(T, H)
    (pos1, pos2, w1, w2, be, nb, load,
     first, par, nxte) = _router(x, W_r, b_r.reshape(1, E))
    xs, ws = _dispatch(x, pos1, pos2, w1, w2)
    y = _ffn(be, nb, first, par, nxte, xs, ws, W1, b1, W2, b2)
    out = _combine(y, pos1, pos2)
    return out.reshape(B, S, H), load.reshape(E)
